# baseline (speedup 1.0000x reference)
import dataclasses
import functools

import jax
import jax.numpy as jnp
from jax import lax
from jax.experimental import pallas as pl
from jax.experimental.pallas import tpu as pltpu
from jax.experimental.pallas import tpu_sc as plsc

LATENT = 32
NUM_WORKERS = 32
CHUNK = 128
LANES = 16


def _make_kernel(batch: int):
    b_per_w = batch // NUM_WORKERS
    n_chunks = b_per_w // CHUNK
    mesh = plsc.VectorSubcoreMesh(core_axis_name="c", subcore_axis_name="s")

    cparams = pltpu.CompilerParams(
        use_tc_tiling_on_sc=False, needs_layout_passes=False
    )

    @functools.partial(
        pl.kernel,
        compiler_params=cparams,
        out_type=jax.ShapeDtypeStruct((batch,), jnp.float32),
        mesh=mesh,
        scratch_types=[
            pltpu.VMEM((n_chunks, CHUNK), jnp.int32),
            pltpu.VMEM((n_chunks, CHUNK), jnp.int32),
            pltpu.VMEM((b_per_w, LATENT), jnp.float32),
            pltpu.VMEM((b_per_w, LATENT), jnp.float32),
            pltpu.VMEM((b_per_w,), jnp.float32),
            pltpu.SemaphoreType.DMA,
            pltpu.SemaphoreType.DMA,
        ],
    )
    def bpr_kernel(user_hbm, item_hbm, u_tab, i_tab, out_hbm,
                   uidx_v, iidx_v, urows_v, irows_v, out_v, idx_sem, sem):
        wid = lax.axis_index("s") * 2 + lax.axis_index("c")
        base = wid * b_per_w

        cp_u = pltpu.async_copy(user_hbm.at[wid], uidx_v, idx_sem)
        cp_i = pltpu.async_copy(item_hbm.at[wid], iidx_v, idx_sem)
        cp_u.wait()
        cp_i.wait()

        copies = []
        for c in range(n_chunks):
            rows = pl.ds(c * CHUNK, CHUNK)
            copies.append(pltpu.async_copy(u_tab.at[uidx_v.at[c]],
                                           urows_v.at[rows], sem))
            copies.append(pltpu.async_copy(i_tab.at[iidx_v.at[c]],
                                           irows_v.at[rows], sem))
        for cp in copies:
            cp.wait()

        lane_iota = lax.broadcasted_iota(jnp.int32, (LANES,), 0)

        @pl.loop(0, b_per_w, step=LANES)
        def _(b):
            row_ids = lane_iota + b
            acc = jnp.zeros((LANES,), jnp.float32)
            for j in range(LATENT):
                col = jnp.full((LANES,), j, jnp.int32)
                uc = plsc.load_gather(urows_v, [row_ids, col])
                ic = plsc.load_gather(irows_v, [row_ids, col])
                acc = acc + uc * ic
            out_v[pl.ds(b, LANES)] = acc

        pltpu.sync_copy(out_v, out_hbm.at[pl.ds(base, b_per_w)])

    return bpr_kernel


def kernel(user, item, U, I):
    batch = user.shape[0]
    b_per_w = batch // NUM_WORKERS
    n_chunks = b_per_w // CHUNK
    uidx = user.reshape(-1).astype(jnp.int32).reshape(NUM_WORKERS, n_chunks, CHUNK)
    iidx = item.reshape(-1).astype(jnp.int32).reshape(NUM_WORKERS, n_chunks, CHUNK)
    out = _make_kernel(batch)(uidx, iidx, U, I)
    return out.reshape(batch, 1)


# speedup vs baseline: 3.4736x; 3.4736x over previous
import functools

import jax
import jax.numpy as jnp
from jax import lax
from jax.experimental import pallas as pl
from jax.experimental.pallas import tpu as pltpu
from jax.experimental.pallas import tpu_sc as plsc

LATENT = 32
NUM_WORKERS = 32
LANES = 16
WAVE = 8


def _make_kernel(batch: int):
    b_per_w = batch // NUM_WORKERS
    n_super = b_per_w // LANES
    mesh = plsc.VectorSubcoreMesh(core_axis_name="c", subcore_axis_name="s")
    cparams = pltpu.CompilerParams(needs_layout_passes=False)

    @functools.partial(
        pl.kernel,
        out_type=jax.ShapeDtypeStruct((batch,), jnp.float32),
        mesh=mesh,
        compiler_params=cparams,
        scratch_types=[
            pltpu.VMEM((b_per_w,), jnp.int32),
            pltpu.VMEM((b_per_w,), jnp.int32),
            pltpu.VMEM((WAVE, LATENT, 128), jnp.float32),
            pltpu.VMEM((WAVE, LATENT, 128), jnp.float32),
            pltpu.VMEM((LANES, LANES), jnp.float32),
            pltpu.VMEM((b_per_w,), jnp.float32),
            pltpu.SemaphoreType.DMA,
            pltpu.SemaphoreType.DMA,
        ],
    )
    def bpr_kernel(user_hbm, item_hbm, ut_hbm, it_hbm, out_hbm,
                   uidx_v, iidx_v, ubuf_v, ibuf_v, tbuf_v, out_v,
                   idx_sem, sem):
        wid = lax.axis_index("s") * 2 + lax.axis_index("c")
        cp_u = pltpu.async_copy(user_hbm.at[wid], uidx_v, idx_sem)
        cp_i = pltpu.async_copy(item_hbm.at[wid], iidx_v, idx_sem)
        cp_u.wait()
        cp_i.wait()

        iota = lax.broadcasted_iota(jnp.int32, (LANES,), 0)
        zeros_i = jnp.zeros((LANES,), jnp.int32)

        @pl.loop(0, n_super)
        def _(sb):
            base = sb * LANES
            uvec = uidx_v[pl.ds(base, LANES)]
            ivec = iidx_v[pl.ds(base, LANES)]
            for half in range(2):
                copies = []
                for kk in range(WAVE):
                    k = half * WAVE + kk
                    bu = uvec[k]
                    bi = ivec[k]
                    su = pl.multiple_of((bu // 128) * 128, 128)
                    si = pl.multiple_of((bi // 128) * 128, 128)
                    copies.append(pltpu.async_copy(
                        ut_hbm.at[:, pl.ds(su, 128)], ubuf_v.at[kk], sem))
                    copies.append(pltpu.async_copy(
                        it_hbm.at[:, pl.ds(si, 128)], ibuf_v.at[kk], sem))
                for cp in copies:
                    cp.wait()
                for kk in range(WAVE):
                    k = half * WAVE + kk
                    lu = zeros_i + (uvec[k] % 128)
                    li = zeros_i + (ivec[k] % 128)
                    u_lo = plsc.load_gather(ubuf_v.at[kk], [iota, lu])
                    u_hi = plsc.load_gather(ubuf_v.at[kk], [iota + 16, lu])
                    i_lo = plsc.load_gather(ibuf_v.at[kk], [iota, li])
                    i_hi = plsc.load_gather(ibuf_v.at[kk], [iota + 16, li])
                    p = u_lo * i_lo + u_hi * i_hi
                    plsc.store_scatter(tbuf_v, [iota, zeros_i + k], p)
            acc = jnp.zeros((LANES,), jnp.float32)
            for r in range(LANES):
                acc = acc + tbuf_v.at[r][...]
            out_v[pl.ds(base, LANES)] = acc

        pltpu.sync_copy(out_v, out_hbm.at[pl.ds(wid * b_per_w, b_per_w)])

    return bpr_kernel


def kernel(user, item, U, I):
    batch = user.shape[0]
    b_per_w = batch // NUM_WORKERS
    uidx = user.reshape(-1).astype(jnp.int32).reshape(NUM_WORKERS, b_per_w)
    iidx = item.reshape(-1).astype(jnp.int32).reshape(NUM_WORKERS, b_per_w)
    out = _make_kernel(batch)(uidx, iidx, U.T, I.T)
    return out.reshape(batch, 1)
